# baseline (speedup 1.0000x reference)
import functools

import jax
import jax.numpy as jnp
from jax import lax
from jax.experimental import pallas as pl
from jax.experimental.pallas import tpu as pltpu
from jax.experimental.pallas import tpu_sc as plsc

MAX_JUSTICES = 100000
NUM_QUERIES = 8
DIM = 64
BATCH = 16384

NUM_CORES = 2
NUM_SUBCORES = 16
NUM_WORKERS = NUM_CORES * NUM_SUBCORES
NTC = MAX_JUSTICES * NUM_QUERIES // 128
TPW = 196
CAP = 3072
SLOTS = 8
LANES = 16

_mesh = plsc.VectorSubcoreMesh(core_axis_name="c", subcore_axis_name="s")


@functools.partial(
    pl.kernel,
    out_type=jax.ShapeDtypeStruct((BATCH, NUM_QUERIES, DIM), jnp.float32),
    mesh=_mesh,
    compiler_params=pltpu.CompilerParams(needs_layout_passes=False),
    scratch_types=[
        pltpu.VMEM((BATCH,), jnp.int32),
        pltpu.VMEM((CAP + LANES,), jnp.int32),
        pltpu.VMEM((CAP + LANES,), jnp.int32),
        pltpu.VMEM((CAP + LANES,), jnp.int32),
        pltpu.VMEM((CAP + LANES,), jnp.int32),
        pltpu.VMEM((224,), jnp.int32),
        pltpu.VMEM((224,), jnp.int32),
        pltpu.VMEM((3, DIM, 128), jnp.float32),
        pltpu.VMEM((SLOTS, NUM_QUERIES, DIM), jnp.float32),
        pltpu.SemaphoreType.DMA,
        pltpu.SemaphoreType.DMA,
        pltpu.SemaphoreType.DMA,
        pltpu.SemaphoreType.DMA,
    ],
)
def _gather_kernel(
    ids_hbm, wt_hbm, out_hbm,
    ids_v, hit_i, hit_id, sort_i, sort_id, sstart, cursor, slab_v, tb_v,
    g0, g1, g2, dsem,
):
    wid = lax.axis_index("s") * NUM_CORES + lax.axis_index("c")
    lo = jnp.minimum(wid * TPW, NTC - TPW)
    hi = lo + TPW

    pltpu.sync_copy(ids_hbm, ids_v)

    lane = lax.iota(jnp.int32, LANES)
    zeros = jnp.zeros((LANES,), jnp.int32)
    ones = zeros + 1

    def scan(g, off):
        v = ids_v[pl.ds(g * LANES, LANES)]
        tc = lax.shift_right_logical(v, 4)
        m = (tc >= lo) & (tc < hi)
        plsc.store_compressed(hit_i.at[pl.ds(off, LANES)], g * LANES + lane, mask=m)
        plsc.store_compressed(hit_id.at[pl.ds(off, LANES)], v, mask=m)
        return off + jnp.max(plsc.all_reduce_population_count(m))

    nhits = lax.fori_loop(0, BATCH // LANES, scan, 0)
    hit_id[pl.ds(nhits, LANES)] = zeros - 1

    for r in range(224 // LANES):
        cursor[pl.ds(r * LANES, LANES)] = zeros

    nvh = lax.div(nhits + LANES - 1, LANES)

    def hist(vh, carry):
        idv = hit_id[pl.ds(vh * LANES, LANES)]
        tcl = lax.shift_right_logical(idv, 4) - lo
        m = (vh * LANES + lane) < nhits
        plsc.addupdate_scatter(cursor, [tcl], ones, mask=m)
        return carry

    lax.fori_loop(0, nvh, hist, 0)

    carry = jnp.int32(0)
    for r in range(224 // LANES):
        v = cursor[pl.ds(r * LANES, LANES)]
        cs = plsc.cumsum(v)
        excl = cs - v + carry
        sstart[pl.ds(r * LANES, LANES)] = excl
        cursor[pl.ds(r * LANES, LANES)] = excl
        carry = carry + jnp.max(cs)

    def scat(h, carry):
        hb = zeros + h
        idv = plsc.load_gather(hit_id, [hb])
        iv = plsc.load_gather(hit_i, [hb])
        tclv = lax.shift_right_logical(idv, 4) - lo
        posv = plsc.load_gather(cursor, [tclv])
        plsc.store_scatter(sort_id, [posv], idv)
        plsc.store_scatter(sort_i, [posv], iv)
        plsc.store_scatter(cursor, [tclv], posv + 1)
        return carry

    lax.fori_loop(0, nhits, scat, 0)

    gsems = (g0, g1, g2)

    def start_slab(t, b):
        pltpu.async_copy(
            wt_hbm.at[:, pl.ds((lo + t) * 128, 128)], slab_v.at[b], gsems[b]
        )

    def wait_slab(b):
        pltpu.make_async_copy(
            wt_hbm.at[:, pl.ds(0, 128)], slab_v.at[b], gsems[b]
        ).wait()

    def wait_out():
        pltpu.make_async_copy(out_hbm.at[0], tb_v.at[0], dsem).wait()

    def stage(t, b, fired):
        wait_slab(b)
        s0 = jnp.max(plsc.load_gather(sstart, [zeros + t]))
        s1 = jnp.max(plsc.load_gather(sstart, [zeros + (t + 1)]))

        def per_hit(h, fr):
            hb = zeros + h
            col0 = (plsc.load_gather(sort_id, [hb]) & 15) * NUM_QUERIES
            i_sc = jnp.max(plsc.load_gather(sort_i, [hb]))
            slot = lax.rem(fr, SLOTS)

            @pl.when(fr >= SLOTS)
            def _():
                wait_out()

            for q in range(NUM_QUERIES):
                for d0 in range(0, DIM, LANES):
                    vv = plsc.load_gather(slab_v.at[b], [d0 + lane, col0 + q])
                    tb_v[slot, q, pl.ds(d0, LANES)] = vv
            pltpu.async_copy(tb_v.at[slot], out_hbm.at[i_sc], dsem)
            return fr + 1

        fired = lax.fori_loop(s0, s1, per_hit, fired)

        @pl.when(t + 3 < TPW)
        def _():
            start_slab(t + 3, b)

        return fired

    start_slab(0, 0)
    start_slab(1, 1)
    start_slab(2, 2)

    def ring_body(t, fr):
        fr = stage(t, 0, fr)
        fr = stage(t + 1, 1, fr)
        fr = stage(t + 2, 2, fr)
        return fr

    fired = pl.loop(0, TPW - 1, step=3, init_carry=jnp.int32(0))(ring_body)
    fired = stage(TPW - 1, 0, fired)

    def drain(k, carry):
        wait_out()
        return carry

    lax.fori_loop(0, jnp.minimum(fired, SLOTS), drain, 0)


def kernel(justice_ids, W):
    return _gather_kernel(justice_ids.astype(jnp.int32), W.T)


# speedup vs baseline: 1.2860x; 1.2860x over previous
import functools

import jax
import jax.numpy as jnp
from jax import lax
from jax.experimental import pallas as pl
from jax.experimental.pallas import tpu as pltpu
from jax.experimental.pallas import tpu_sc as plsc

MAX_JUSTICES = 100000
NUM_QUERIES = 8
DIM = 64
BATCH = 16384

NUM_CORES = 2
NUM_SUBCORES = 16
NUM_WORKERS = NUM_CORES * NUM_SUBCORES
IDS_PER_WORKER = BATCH // NUM_WORKERS
CHUNK = 32
NUM_CHUNKS = IDS_PER_WORKER // CHUNK
LANES = 16

_mesh = plsc.VectorSubcoreMesh(core_axis_name="c", subcore_axis_name="s")


@functools.partial(
    pl.kernel,
    out_type=jax.ShapeDtypeStruct((BATCH * NUM_QUERIES, DIM), jnp.float32),
    mesh=_mesh,
    compiler_params=pltpu.CompilerParams(needs_layout_passes=False),
    scratch_types=[
        pltpu.VMEM((IDS_PER_WORKER,), jnp.int32),
        pltpu.VMEM((3, CHUNK * NUM_QUERIES, DIM), jnp.float32),
        pltpu.SemaphoreType.DMA,
        pltpu.SemaphoreType.DMA,
        pltpu.SemaphoreType.DMA,
        pltpu.SemaphoreType.DMA,
        pltpu.SemaphoreType.DMA,
        pltpu.SemaphoreType.DMA,
    ],
)
def _gather_kernel(
    ids_hbm, table_hbm, out_hbm, ids_v, blocks_v, g0, g1, g2, d0, d1, d2
):
    wid = lax.axis_index("s") * NUM_CORES + lax.axis_index("c")
    base = wid * IDS_PER_WORKER

    pltpu.sync_copy(ids_hbm.at[pl.ds(base, IDS_PER_WORKER)], ids_v)

    lane = lax.iota(jnp.int32, LANES)
    gsems = (g0, g1, g2)
    dsems = (d0, d1, d2)

    def start_chunk(c, buf):
        @pl.loop(0, CHUNK // LANES)
        def per_group(g):
            ids16 = ids_v[pl.ds((c * CHUNK + g * LANES), LANES)]
            for l in range(LANES):
                row = jnp.max(jnp.where(lane == l, ids16, -1))
                pltpu.async_copy(
                    table_hbm.at[row],
                    blocks_v.at[buf].at[pl.ds((g * LANES + l) * NUM_QUERIES, NUM_QUERIES)],
                    gsems[buf],
                )

    def wait_sem(buf, sem):
        pltpu.make_async_copy(
            out_hbm.at[pl.ds(0, CHUNK * NUM_QUERIES)], blocks_v.at[buf], sem
        ).wait()

    def stage(c, buf):
        wait_sem(buf, gsems[buf])
        pltpu.async_copy(
            blocks_v.at[buf],
            out_hbm.at[pl.ds((base + c * CHUNK) * NUM_QUERIES, CHUNK * NUM_QUERIES)],
            dsems[buf],
        )
        nxt = (buf + 2) % 3

        @pl.when(c + 2 < NUM_CHUNKS)
        def _():
            @pl.when(c >= 1)
            def _():
                wait_sem(nxt, dsems[nxt])

            start_chunk(c + 2, nxt)

    start_chunk(0, 0)
    start_chunk(1, 1)

    @pl.loop(0, NUM_CHUNKS - 1, step=3)
    def ring(c):
        stage(c, 0)
        stage(c + 1, 1)
        stage(c + 2, 2)

    stage(NUM_CHUNKS - 1, 0)
    wait_sem(0, d0)
    wait_sem(1, d1)
    wait_sem(2, d2)


def kernel(justice_ids, W):
    table = W.reshape(MAX_JUSTICES, NUM_QUERIES, DIM)
    out = _gather_kernel(justice_ids.astype(jnp.int32), table)
    return out.reshape(BATCH, NUM_QUERIES, DIM)
